# baseline (speedup 1.0000x reference)
import functools

import jax
import jax.numpy as jnp
from jax import lax
from jax.experimental import pallas as pl
from jax.experimental.pallas import tpu as pltpu
from jax.experimental.pallas import tpu_sc as plsc

N = 10000
E = 320000
D = 128
H = 128
O = 64
G = 8

NP = 10240
RP = NP // 16
NW = 32
EW = E // NW
C = 80
NCH = EW // C
CM = 40
NCM = EW // CM

_f32 = jnp.float32
_mesh = plsc.VectorSubcoreMesh(core_axis_name="c", subcore_axis_name="s")


def _zero16():
    return jnp.zeros((16,), _f32)


@functools.partial(
    pl.kernel,
    out_type=jax.ShapeDtypeStruct((2, NP), _f32),
    mesh=_mesh,
    scratch_types=[
        pltpu.VMEM((NCH, C), jnp.int32),
        pltpu.VMEM((NCH, C), _f32),
        pltpu.VMEM((RP,), _f32),
        pltpu.VMEM_SHARED((NP,), _f32),
    ],
)
def _deg_kernel(col_hbm, w_hbm, out_hbm, cib, wb, dbounce, dacc):
    cid = lax.axis_index("c")
    sid = lax.axis_index("s")
    gwid = sid * 2 + cid

    for i in range(RP // 16):
        dbounce[pl.ds(i * 16, 16)] = _zero16()
    pltpu.sync_copy(dbounce, dacc.at[pl.ds(sid * RP, RP)])
    pltpu.sync_copy(col_hbm.at[gwid], cib)
    pltpu.sync_copy(w_hbm.at[gwid], wb)
    plsc.subcore_barrier()

    def chunk(g, carry):
        pltpu.sync_copy(wb.at[g], dacc.at[cib.at[g]], add=True)
        return carry

    lax.fori_loop(0, NCH, chunk, 0)
    plsc.subcore_barrier()

    pltpu.sync_copy(dacc.at[pl.ds(sid * RP, RP)],
                    out_hbm.at[cid, pl.ds(sid * RP, RP)])


_NGB = 4
_NPK = 8
_UNR = 8
_NTL = NCM - (NCM // _UNR) * _UNR


@functools.partial(
    pl.kernel,
    out_type=jax.ShapeDtypeStruct((2, NP, D), _f32),
    mesh=_mesh,
    scratch_types=(
        [pltpu.VMEM((4, CM), jnp.int32)] * _NPK
        + [pltpu.VMEM((CM, D), _f32)] * _NGB
        + [pltpu.VMEM_SHARED((NP, D), _f32)]
        + [pltpu.SemaphoreType.DMA] * (_NPK + 2 * _NGB)
    ),
)
def _msg_kernel(xs_hbm, pk_hbm, out_hbm, *refs):
    pkb = refs[:_NPK]
    gbufs = refs[_NPK:_NPK + _NGB]
    acc = refs[_NPK + _NGB]
    sems = refs[_NPK + _NGB + 1:]
    pks = sems[:_NPK]
    gsems = sems[_NPK:_NPK + _NGB]
    ssems = sems[_NPK + _NGB:]

    cid = lax.axis_index("c")
    sid = lax.axis_index("s")
    gwid = sid * 2 + cid
    gb0 = gbufs[0]

    def zrow(r, carry):
        for k in range(D // 16):
            gb0[r, pl.ds(k * 16, 16)] = _zero16()
        return carry

    lax.fori_loop(0, CM, zrow, 0)
    for k in range(RP // CM):
        pltpu.sync_copy(gb0, acc.at[pl.ds(sid * RP + k * CM, CM)])
    plsc.subcore_barrier()

    def start_pk(q, r):
        pltpu.async_copy(pk_hbm.at[gwid, q], pkb[r], pks[r])

    def wait_pk(q, r):
        pltpu.make_async_copy(pk_hbm.at[gwid, q], pkb[r], pks[r]).wait()

    def start_ga(q, r, b):
        pltpu.async_copy(xs_hbm.at[pkb[r].at[0]], gbufs[b], gsems[b])

    def wait_ga(q, r, b):
        pltpu.make_async_copy(xs_hbm.at[pkb[r].at[0]], gbufs[b],
                              gsems[b]).wait()

    def start_sc(q, r, b):
        pltpu.async_copy(gbufs[b], acc.at[pkb[r].at[1]], ssems[b], add=True)

    def wait_sc(q, r, b):
        pltpu.make_async_copy(gbufs[b], acc.at[pkb[r].at[1]],
                              ssems[b]).wait()

    def scale(q, r, b):
        gb = gbufs[b]
        pk = pkb[r]

        def body(grp, c2):
            wv = lax.bitcast_convert_type(pk[2, pl.ds(grp * 16, 16)], _f32)
            for jj in range(16):
                e = grp * 16 + jj
                ws = wv[jj]
                for k in range(D // 16):
                    gb[e, pl.ds(k * 16, 16)] = gb[e, pl.ds(k * 16, 16)] * ws
            return c2

        lax.fori_loop(0, CM // 16, body, 0)
        wv = lax.bitcast_convert_type(pk[2, pl.ds(CM - 16, 16)], _f32)
        for jj in range(8, 16):
            e = CM - 16 + jj
            ws = wv[jj]
            for k in range(D // 16):
                gb[e, pl.ds(k * 16, 16)] = gb[e, pl.ds(k * 16, 16)] * ws

    def step(q, j, cond):
        b = j % _NGB
        r = j % _NPK
        wait_ga(q, r, b)
        scale(q, r, b)
        start_sc(q, r, b)
        cond(q >= 2, lambda: wait_sc(q - 2, (j - 2) % _NPK, (j - 2) % _NGB))
        cond(q + 2 < NCM,
             lambda: (wait_pk(q + 2, (j + 2) % _NPK),
                      start_ga(q + 2, (j + 2) % _NPK, (j + 2) % _NGB)))
        cond(q + 3 < NCM, lambda: start_pk(q + 3, (j + 3) % _NPK))

    start_pk(0, 0)
    start_pk(1, 1)
    start_pk(2, 2)
    wait_pk(0, 0)
    start_ga(0, 0, 0)
    wait_pk(1, 1)
    start_ga(1, 1, 1)

    def traced_cond(pred, fn):
        pl.when(pred)(lambda: (fn(), None)[1])

    def eight(go, carry):
        g = go * _UNR
        for j in range(_UNR):
            step(g + j, j, traced_cond)
        return carry

    nfull = NCM // _UNR
    lax.fori_loop(0, nfull, eight, 0)

    def static_cond(pred, fn):
        if pred:
            fn()

    for j in range(_NTL):
        step(nfull * _UNR + j, j, static_cond)

    for q in (NCM - 2, NCM - 1):
        wait_sc(q, (q % _UNR) % _NPK, (q % _UNR) % _NGB)
    plsc.subcore_barrier()

    pltpu.sync_copy(acc.at[pl.ds(sid * RP, RP)],
                    out_hbm.at[cid, pl.ds(sid * RP, RP)])


def _t1_body(x_ref, w1_ref, d0_ref, d1_ref, xs_ref, dinv_ref):
    deg = d0_ref[...] + d1_ref[...] + 1.0
    dinv = lax.rsqrt(deg)
    xw = jnp.dot(x_ref[...], w1_ref[...], preferred_element_type=_f32,
                 precision=lax.Precision.HIGHEST)
    xs_ref[...] = xw * dinv
    dinv_ref[...] = dinv


def _t2_body(accp_ref, xs_ref, dinv_ref, b1_ref, gamma_ref, beta_ref, w2_ref,
             hs2_ref):
    acc = accp_ref[0, :N, :] + accp_ref[1, :N, :]
    dinv = dinv_ref[...]
    pre = (acc + xs_ref[...]) * dinv + b1_ref[...]
    mean = jnp.mean(pre, axis=0, keepdims=True)
    cen = pre - mean
    var = jnp.mean(cen * cen, axis=0, keepdims=True)
    h = cen * lax.rsqrt(var + 1e-5) * gamma_ref[...] + beta_ref[...]
    h = jnp.maximum(h, 0.0)
    hs2_ref[...] = jnp.dot(h, w2_ref[...], preferred_element_type=_f32,
                           precision=lax.Precision.HIGHEST) * dinv


_BT = 1000


def _t3_body(acc2_ref, hs2_ref, dinv_ref, b2_ref, batch_ref, wl_ref, bl_ref,
             out_ref):
    i = pl.program_id(0)
    acc = acc2_ref[0] + acc2_ref[1]
    h2 = (acc + hs2_ref[...]) * dinv_ref[...] + b2_ref[...]
    onehot = (batch_ref[...] == lax.broadcasted_iota(jnp.int32, (_BT, G), 1))
    onehot = onehot.astype(_f32)
    pooled = lax.dot_general(onehot, h2, (((0,), (0,)), ((), ())),
                             preferred_element_type=_f32,
                             precision=lax.Precision.HIGHEST)
    part = jnp.dot(pooled, wl_ref[...], preferred_element_type=_f32,
                   precision=lax.Precision.HIGHEST)

    @pl.when(i == 0)
    def _():
        out_ref[...] = jnp.broadcast_to(bl_ref[...], (G, O))

    out_ref[...] += part


_t1 = pl.pallas_call(
    _t1_body,
    out_shape=[jax.ShapeDtypeStruct((N, D), _f32),
               jax.ShapeDtypeStruct((N, 1), _f32)],
)

_t2 = pl.pallas_call(
    _t2_body,
    out_shape=jax.ShapeDtypeStruct((N, H), _f32),
)

def _t3s_body(acc2_ref, hs2_ref, dinv_ref, b2_ref, batch_ref, wl_ref, bl_ref,
              out_ref):
    acc = acc2_ref[0, :N, :] + acc2_ref[1, :N, :]
    h2 = (acc + hs2_ref[...]) * dinv_ref[...] + b2_ref[...]
    onehot = (batch_ref[...] == lax.broadcasted_iota(jnp.int32, (N, G), 1))
    onehot = onehot.astype(_f32)
    pooled = lax.dot_general(onehot, h2, (((0,), (0,)), ((), ())),
                             preferred_element_type=_f32,
                             precision=lax.Precision.HIGHEST)
    out_ref[...] = jnp.dot(pooled, wl_ref[...], preferred_element_type=_f32,
                           precision=lax.Precision.HIGHEST) + bl_ref[...]


_t3 = pl.pallas_call(
    _t3s_body,
    out_shape=jax.ShapeDtypeStruct((G, O), _f32),
)


def kernel(x, edge_index, edge_weights, batch, W1, b1, gamma, beta, W2, b2,
           Wl, bl):
    col = edge_index[1].reshape(NW, NCH, C)
    ew = edge_weights.reshape(NW, NCH, C)
    rowm = edge_index[0].reshape(NW, NCM, CM)
    colm = edge_index[1].reshape(NW, NCM, CM)
    wbits = lax.bitcast_convert_type(edge_weights, jnp.int32).reshape(
        NW, NCM, CM)
    pk = jnp.stack([rowm, colm, wbits, wbits], axis=2)

    degp = _deg_kernel(col, ew)
    d0 = degp[0, :N].reshape(N, 1)
    d1 = degp[1, :N].reshape(N, 1)

    xs, dinv = _t1(x, W1, d0, d1)
    accp = _msg_kernel(xs, pk)
    hs2 = _t2(accp, xs, dinv, b1.reshape(1, H), gamma.reshape(1, H),
              beta.reshape(1, H), W2)
    acc2p = _msg_kernel(hs2, pk)
    out = _t3(acc2p, hs2, dinv, b2.reshape(1, H), batch.reshape(N, 1), Wl,
              bl.reshape(1, O))
    return out


# speedup vs baseline: 1.2774x; 1.2774x over previous
import functools

import jax
import jax.numpy as jnp
from jax import lax
from jax.experimental import pallas as pl
from jax.experimental.pallas import tpu as pltpu
from jax.experimental.pallas import tpu_sc as plsc

N = 10000
E = 320000
D = 128
H = 128
O = 64
G = 8

NP = 10240
RP = NP // 16
NW = 32
EW = E // NW
C = 80
NCH = EW // C

_f32 = jnp.float32
_mesh = plsc.VectorSubcoreMesh(core_axis_name="c", subcore_axis_name="s")


def _zero16():
    return jnp.zeros((16,), _f32)


@functools.partial(
    pl.kernel,
    out_type=jax.ShapeDtypeStruct((2, NP), _f32),
    mesh=_mesh,
    scratch_types=[
        pltpu.VMEM((NCH, C), jnp.int32),
        pltpu.VMEM((NCH, C), _f32),
        pltpu.VMEM((RP,), _f32),
        pltpu.VMEM_SHARED((NP,), _f32),
    ],
)
def _deg_kernel(col_hbm, w_hbm, out_hbm, cib, wb, dbounce, dacc):
    cid = lax.axis_index("c")
    sid = lax.axis_index("s")
    gwid = sid * 2 + cid

    for i in range(RP // 16):
        dbounce[pl.ds(i * 16, 16)] = _zero16()
    pltpu.sync_copy(dbounce, dacc.at[pl.ds(sid * RP, RP)])
    pltpu.sync_copy(col_hbm.at[gwid], cib)
    pltpu.sync_copy(w_hbm.at[gwid], wb)
    plsc.subcore_barrier()

    def chunk(g, carry):
        pltpu.sync_copy(wb.at[g], dacc.at[cib.at[g]], add=True)
        return carry

    lax.fori_loop(0, NCH, chunk, 0)
    plsc.subcore_barrier()

    pltpu.sync_copy(dacc.at[pl.ds(sid * RP, RP)],
                    out_hbm.at[cid, pl.ds(sid * RP, RP)])


_NGB = 3
_NPK = 6
_UNR = 6
_NTL = NCH - (NCH // _UNR) * _UNR


@functools.partial(
    pl.kernel,
    out_type=jax.ShapeDtypeStruct((2, NP, D), _f32),
    mesh=_mesh,
    scratch_types=(
        [pltpu.VMEM((3, C), jnp.int32)] * _NPK
        + [pltpu.VMEM((C, D), _f32)] * _NGB
        + [pltpu.VMEM_SHARED((NP, D), _f32)]
        + [pltpu.SemaphoreType.DMA] * (_NPK + 2 * _NGB)
    ),
)
def _msg_kernel(xs_hbm, pk_hbm, out_hbm, *refs):
    pkb = refs[:_NPK]
    gbufs = refs[_NPK:_NPK + _NGB]
    acc = refs[_NPK + _NGB]
    sems = refs[_NPK + _NGB + 1:]
    pks = sems[:_NPK]
    gsems = sems[_NPK:_NPK + _NGB]
    ssems = sems[_NPK + _NGB:]

    cid = lax.axis_index("c")
    sid = lax.axis_index("s")
    gwid = sid * 2 + cid
    gb0 = gbufs[0]

    def zrow(r, carry):
        for k in range(D // 16):
            gb0[r, pl.ds(k * 16, 16)] = _zero16()
        return carry

    lax.fori_loop(0, C, zrow, 0)
    for k in range(RP // C):
        pltpu.sync_copy(gb0, acc.at[pl.ds(sid * RP + k * C, C)])
    plsc.subcore_barrier()

    def start_pk(q, r):
        pltpu.async_copy(pk_hbm.at[gwid, q], pkb[r], pks[r])

    def wait_pk(q, r):
        pltpu.make_async_copy(pk_hbm.at[gwid, q], pkb[r], pks[r]).wait()

    def start_ga(q, r, b):
        pltpu.async_copy(xs_hbm.at[pkb[r].at[0]], gbufs[b], gsems[b])

    def wait_ga(q, r, b):
        pltpu.make_async_copy(xs_hbm.at[pkb[r].at[0]], gbufs[b],
                              gsems[b]).wait()

    def start_sc(q, r, b):
        pltpu.async_copy(gbufs[b], acc.at[pkb[r].at[1]], ssems[b], add=True)

    def wait_sc(q, r, b):
        pltpu.make_async_copy(gbufs[b], acc.at[pkb[r].at[1]],
                              ssems[b]).wait()

    def scale(q, r, b):
        gb = gbufs[b]
        pk = pkb[r]

        def body(grp, c2):
            wv = lax.bitcast_convert_type(pk[2, pl.ds(grp * 16, 16)], _f32)
            for jj in range(16):
                e = grp * 16 + jj
                ws = wv[jj]
                for k in range(D // 16):
                    gb[e, pl.ds(k * 16, 16)] = gb[e, pl.ds(k * 16, 16)] * ws
            return c2

        lax.fori_loop(0, C // 16, body, 0)

    def step(q, j, cond):
        b = j % _NGB
        r = j % _NPK
        wait_ga(q, r, b)
        scale(q, r, b)
        start_sc(q, r, b)
        cond(q >= 1, lambda: wait_sc(q - 1, (j - 1) % _NPK, (j - 1) % _NGB))
        cond(q + 2 < NCH,
             lambda: (wait_pk(q + 2, (j + 2) % _NPK),
                      start_ga(q + 2, (j + 2) % _NPK, (j + 2) % _NGB)))
        cond(q + 3 < NCH, lambda: start_pk(q + 3, (j + 3) % _NPK))

    start_pk(0, 0)
    start_pk(1, 1)
    start_pk(2, 2)
    wait_pk(0, 0)
    start_ga(0, 0, 0)
    wait_pk(1, 1)
    start_ga(1, 1, 1)

    def traced_cond(pred, fn):
        pl.when(pred)(lambda: (fn(), None)[1])

    def six(go, carry):
        g = go * _UNR
        for j in range(_UNR):
            step(g + j, j, traced_cond)
        return carry

    nfull = NCH // _UNR
    lax.fori_loop(0, nfull, six, 0)

    def static_cond(pred, fn):
        if pred:
            fn()

    for j in range(_NTL):
        step(nfull * _UNR + j, j, static_cond)

    qlast = NCH - 1
    wait_sc(qlast, (qlast % _UNR) % _NPK, (qlast % _UNR) % _NGB)
    plsc.subcore_barrier()

    pltpu.sync_copy(acc.at[pl.ds(sid * RP, RP)],
                    out_hbm.at[cid, pl.ds(sid * RP, RP)])


def _t1_body(x_ref, w1_ref, d0_ref, d1_ref, xs_ref, dinv_ref):
    deg = d0_ref[...] + d1_ref[...] + 1.0
    dinv = lax.rsqrt(deg)
    xw = jnp.dot(x_ref[...], w1_ref[...], preferred_element_type=_f32,
                 precision=lax.Precision.HIGHEST)
    xs_ref[...] = xw * dinv
    dinv_ref[...] = dinv


def _t2_body(accp_ref, xs_ref, dinv_ref, b1_ref, gamma_ref, beta_ref, w2_ref,
             hs2_ref):
    acc = accp_ref[0, :N, :] + accp_ref[1, :N, :]
    dinv = dinv_ref[...]
    pre = (acc + xs_ref[...]) * dinv + b1_ref[...]
    mean = jnp.mean(pre, axis=0, keepdims=True)
    cen = pre - mean
    var = jnp.mean(cen * cen, axis=0, keepdims=True)
    h = cen * lax.rsqrt(var + 1e-5) * gamma_ref[...] + beta_ref[...]
    h = jnp.maximum(h, 0.0)
    hs2_ref[...] = jnp.dot(h, w2_ref[...], preferred_element_type=_f32,
                           precision=lax.Precision.HIGHEST) * dinv


_BT = 1000


def _t3_body(acc2_ref, hs2_ref, dinv_ref, b2_ref, batch_ref, wl_ref, bl_ref,
             out_ref):
    i = pl.program_id(0)
    acc = acc2_ref[0] + acc2_ref[1]
    h2 = (acc + hs2_ref[...]) * dinv_ref[...] + b2_ref[...]
    onehot = (batch_ref[...] == lax.broadcasted_iota(jnp.int32, (_BT, G), 1))
    onehot = onehot.astype(_f32)
    pooled = lax.dot_general(onehot, h2, (((0,), (0,)), ((), ())),
                             preferred_element_type=_f32,
                             precision=lax.Precision.HIGHEST)
    part = jnp.dot(pooled, wl_ref[...], preferred_element_type=_f32,
                   precision=lax.Precision.HIGHEST)

    @pl.when(i == 0)
    def _():
        out_ref[...] = jnp.broadcast_to(bl_ref[...], (G, O))

    out_ref[...] += part


_t1 = pl.pallas_call(
    _t1_body,
    out_shape=[jax.ShapeDtypeStruct((N, D), _f32),
               jax.ShapeDtypeStruct((N, 1), _f32)],
)

_t2 = pl.pallas_call(
    _t2_body,
    out_shape=jax.ShapeDtypeStruct((N, H), _f32),
)

def _t3s_body(acc2_ref, hs2_ref, dinv_ref, b2_ref, batch_ref, wl_ref, bl_ref,
              out_ref):
    acc = acc2_ref[0, :N, :] + acc2_ref[1, :N, :]
    h2 = (acc + hs2_ref[...]) * dinv_ref[...] + b2_ref[...]
    onehot = (batch_ref[...] == lax.broadcasted_iota(jnp.int32, (N, G), 1))
    onehot = onehot.astype(_f32)
    pooled = lax.dot_general(onehot, h2, (((0,), (0,)), ((), ())),
                             preferred_element_type=_f32,
                             precision=lax.Precision.HIGHEST)
    out_ref[...] = jnp.dot(pooled, wl_ref[...], preferred_element_type=_f32,
                           precision=lax.Precision.HIGHEST) + bl_ref[...]


_t3 = pl.pallas_call(
    _t3s_body,
    out_shape=jax.ShapeDtypeStruct((G, O), _f32),
)


def kernel(x, edge_index, edge_weights, batch, W1, b1, gamma, beta, W2, b2,
           Wl, bl):
    row = edge_index[0].reshape(NW, NCH, C)
    col = edge_index[1].reshape(NW, NCH, C)
    ew = edge_weights.reshape(NW, NCH, C)
    wbits = lax.bitcast_convert_type(ew, jnp.int32)
    pk = jnp.stack([row, col, wbits], axis=2)

    degp = _deg_kernel(col, ew)
    d0 = degp[0, :N].reshape(N, 1)
    d1 = degp[1, :N].reshape(N, 1)

    xs, dinv = _t1(x, W1, d0, d1)
    accp = _msg_kernel(xs, pk)
    hs2 = _t2(accp, xs, dinv, b1.reshape(1, H), gamma.reshape(1, H),
              beta.reshape(1, H), W2)
    acc2p = _msg_kernel(hs2, pk)
    out = _t3(acc2p, hs2, dinv, b2.reshape(1, H), batch.reshape(N, 1), Wl,
              bl.reshape(1, O))
    return out
